# baseline (speedup 1.0000x reference)
import functools

import jax
import jax.numpy as jnp
from jax import lax
from jax.experimental import pallas as pl
from jax.experimental.pallas import tpu as pltpu
from jax.experimental.pallas import tpu_sc as plsc

_N = 10000
_E = 160000
_F_IN = 256
_F_OUT = 256
_H = 128
_NC = 2
_NS = 16
_EPT = _E // _NS
_K = 80
_NCH = _EPT // _K
_RPT = 624
_LANES = 16

_BCAST_DNUMS = lax.GatherDimensionNumbers(
    offset_dims=(), collapsed_slice_dims=(0,), start_index_map=(0,))


def _matmul_body(x_ref, w_ref, o_ref):
    o_ref[...] = jnp.dot(x_ref[...], w_ref[...],
                         preferred_element_type=jnp.float32)


def _matmul(x, w):
    return pl.pallas_call(
        _matmul_body,
        grid=(_NC,),
        in_specs=[
            pl.BlockSpec((_N, _F_IN), lambda n: (0, 0)),
            pl.BlockSpec((_F_IN, _H), lambda n: (0, n)),
        ],
        out_specs=pl.BlockSpec((_N, _H), lambda n: (n, 0)),
        out_shape=jax.ShapeDtypeStruct((_NC * _N, _H), jnp.float32),
    )(x, w)


def _spmm_body(xws, colr, rowr, adjr, zeros, out,
               col_v, row_v, adjb, rows_v, acc, gsem, asem, ssem):
    c = lax.axis_index("c")
    s = lax.axis_index("s")
    w = c * _NS + s
    r0 = s * _RPT

    pltpu.sync_copy(zeros.at[pl.ds(0, _RPT)], acc.at[pl.ds(r0, _RPT)])

    @pl.when(s == _NS - 1)
    def _():
        rem = _N - _NS * _RPT
        pltpu.sync_copy(zeros.at[pl.ds(0, rem)],
                        acc.at[pl.ds(_NS * _RPT, rem)])
    pltpu.sync_copy(colr.at[pl.ds(w * _EPT, _EPT)], col_v)
    pltpu.sync_copy(rowr.at[s], row_v)
    plsc.subcore_barrier()

    a0 = s * _EPT

    def start_chunk(j, b):
        pltpu.async_copy(xws.at[col_v.at[pl.ds(j * _K, _K)]],
                         rows_v.at[b], gsem)
        pltpu.async_copy(adjr.at[pl.ds(a0 + j * _K, _K)], adjb.at[b], asem)

    start_chunk(0, 0)

    def chunk_body(j, carry):
        b = lax.rem(j, 2)
        bn = 1 - b
        pltpu.make_async_copy(xws.at[col_v.at[pl.ds(j * _K, _K)]],
                              rows_v.at[b], gsem).wait()
        pltpu.make_async_copy(adjr.at[pl.ds(a0, _K)], adjb.at[b],
                              asem).wait()

        @pl.when(j > 0)
        def _():
            pltpu.make_async_copy(rows_v.at[bn], acc.at[pl.ds(0, _K)],
                                  ssem).wait()

        @pl.when(j < _NCH - 1)
        def _():
            start_chunk(j + 1, bn)

        def group_body(g, carry2):
            av = adjb[b, pl.ds(g * _LANES, _LANES)]
            for t in range(_LANES):
                a = lax.gather(
                    av,
                    jnp.full((_LANES, 1), t, dtype=jnp.int32),
                    _BCAST_DNUMS,
                    slice_sizes=(1,),
                    mode=lax.GatherScatterMode.PROMISE_IN_BOUNDS,
                )
                e = g * _LANES + t
                for f in range(_H // _LANES):
                    seg = rows_v[b, e, pl.ds(f * _LANES, _LANES)]
                    rows_v[b, e, pl.ds(f * _LANES, _LANES)] = seg * a
            return carry2

        lax.fori_loop(0, _K // _LANES, group_body, 0)

        pltpu.async_copy(rows_v.at[b], acc.at[row_v.at[j]], ssem, add=True)
        return carry

    lax.fori_loop(0, _NCH, chunk_body, 0)
    pltpu.make_async_copy(rows_v.at[(_NCH - 1) % 2],
                          acc.at[pl.ds(0, _K)], ssem).wait()
    plsc.subcore_barrier()

    pltpu.sync_copy(acc.at[pl.ds(r0, _RPT)],
                    out.at[pl.ds(c * _N + r0, _RPT)])

    @pl.when(s == _NS - 1)
    def _():
        rem = _N - _NS * _RPT
        pltpu.sync_copy(acc.at[pl.ds(_NS * _RPT, rem)],
                        out.at[pl.ds(c * _N + _NS * _RPT, rem)])


_spmm = functools.partial(
    pl.kernel,
    out_type=jax.ShapeDtypeStruct((_NC * _N, _H), jnp.float32),
    mesh=plsc.VectorSubcoreMesh(core_axis_name="c", subcore_axis_name="s"),
    scratch_types=[
        pltpu.VMEM((_EPT,), jnp.int32),
        pltpu.VMEM((_NCH, _K), jnp.int32),
        pltpu.VMEM((2, _K), jnp.float32),
        pltpu.VMEM((2, _K, _H), jnp.float32),
        pltpu.VMEM_SHARED((_N, _H), jnp.float32),
        pltpu.SemaphoreType.DMA,
        pltpu.SemaphoreType.DMA,
        pltpu.SemaphoreType.DMA,
    ],
)(_spmm_body)


def kernel(x, edge_index, adj_values, W, bias):
    row = edge_index[0]
    col = edge_index[1]

    xws = _matmul(x, W)

    colr = jnp.concatenate([col, col + _N])
    rowr = row.reshape(_NS, _NCH, _K)
    adjr = adj_values
    zeros = jnp.zeros((_RPT + 16, _H), dtype=jnp.float32)

    outs = _spmm(xws, colr, rowr, adjr, zeros)

    out = outs.reshape(_NC, _N, _H).transpose(1, 0, 2).reshape(_N, _F_OUT)
    return out + bias


# speedup vs baseline: 2.5883x; 2.5883x over previous
import functools

import jax
import jax.numpy as jnp
from jax import lax
from jax.experimental import pallas as pl
from jax.experimental.pallas import tpu as pltpu
from jax.experimental.pallas import tpu_sc as plsc

_N = 10000
_E = 160000
_F_IN = 256
_F_OUT = 256
_H = 128
_NC = 2
_NS = 16
_EPT = _E // _NS
_K = 80
_NCH = _EPT // _K
_RPT = 624
_LANES = 16

_BCAST_DNUMS = lax.GatherDimensionNumbers(
    offset_dims=(), collapsed_slice_dims=(0,), start_index_map=(0,))


def _matmul_body(x_ref, w_ref, o_ref):
    o_ref[...] = jnp.dot(x_ref[...], w_ref[...],
                         preferred_element_type=jnp.float32)


def _matmul(x, w):
    return pl.pallas_call(
        _matmul_body,
        grid=(_NC,),
        in_specs=[
            pl.BlockSpec((_N, _F_IN), lambda n: (0, 0)),
            pl.BlockSpec((_F_IN, _H), lambda n: (0, n)),
        ],
        out_specs=pl.BlockSpec((_N, _H), lambda n: (n, 0)),
        out_shape=jax.ShapeDtypeStruct((_NC * _N, _H), jnp.float32),
    )(x, w)


def _spmm_body(xws, colr, rowr, adjr, zeros, out,
               col_v, adj_v, rowb, rows_v, acc, gsem, rsem):
    c = lax.axis_index("c")
    s = lax.axis_index("s")
    w = c * _NS + s
    r0 = s * _RPT

    pltpu.sync_copy(zeros.at[pl.ds(0, _RPT)], acc.at[pl.ds(r0, _RPT)])

    @pl.when(s == _NS - 1)
    def _():
        rem = _N - _NS * _RPT
        pltpu.sync_copy(zeros.at[pl.ds(0, rem)],
                        acc.at[pl.ds(_NS * _RPT, rem)])
    pltpu.sync_copy(colr.at[pl.ds(w * _EPT, _EPT)], col_v)
    pltpu.sync_copy(adjr.at[pl.ds(s * _EPT, _EPT)], adj_v)
    plsc.subcore_barrier()

    def start_chunk(j, b):
        pltpu.async_copy(xws.at[col_v.at[pl.ds(j * _K, _K)]],
                         rows_v.at[b], gsem)
        pltpu.async_copy(rowr.at[pl.ds(s * _EPT + j * _K, _K)],
                         rowb.at[b], rsem)

    start_chunk(0, 0)

    def do_chunk(j, b):
        @pl.when(j < _NCH - 1)
        def _():
            start_chunk(j + 1, 1 - b)

        pltpu.make_async_copy(xws.at[col_v.at[pl.ds(j * _K, _K)]],
                              rows_v.at[b], gsem).wait()
        pltpu.make_async_copy(rowr.at[pl.ds(s * _EPT, _K)], rowb.at[b],
                              rsem).wait()

        def group_body(g, carry2):
            av = adj_v[pl.ds(j * _K + g * _LANES, _LANES)]
            for t in range(_LANES):
                a = lax.gather(
                    av,
                    jnp.full((_LANES, 1), t, dtype=jnp.int32),
                    _BCAST_DNUMS,
                    slice_sizes=(1,),
                    mode=lax.GatherScatterMode.PROMISE_IN_BOUNDS,
                )
                e = g * _LANES + t
                for f in range(_H // _LANES):
                    seg = rows_v[b, e, pl.ds(f * _LANES, _LANES)]
                    rows_v[b, e, pl.ds(f * _LANES, _LANES)] = seg * a
            return carry2

        lax.fori_loop(0, _K // _LANES, group_body, 0)

        pltpu.sync_copy(rows_v.at[b], acc.at[rowb.at[b]], add=True)

    def pair_body(t, carry):
        do_chunk(2 * t, 0)
        do_chunk(2 * t + 1, 1)
        return carry

    lax.fori_loop(0, _NCH // 2, pair_body, 0)
    do_chunk(_NCH - 1, 0)
    plsc.subcore_barrier()

    pltpu.sync_copy(acc.at[pl.ds(r0, _RPT)],
                    out.at[pl.ds(c * _N + r0, _RPT)])

    @pl.when(s == _NS - 1)
    def _():
        rem = _N - _NS * _RPT
        pltpu.sync_copy(acc.at[pl.ds(_NS * _RPT, rem)],
                        out.at[pl.ds(c * _N + _NS * _RPT, rem)])


_spmm = functools.partial(
    pl.kernel,
    out_type=jax.ShapeDtypeStruct((_NC * _N, _H), jnp.float32),
    mesh=plsc.VectorSubcoreMesh(core_axis_name="c", subcore_axis_name="s"),
    scratch_types=[
        pltpu.VMEM((_EPT,), jnp.int32),
        pltpu.VMEM((_EPT,), jnp.float32),
        pltpu.VMEM((2, _K), jnp.int32),
        pltpu.VMEM((2, _K, _H), jnp.float32),
        pltpu.VMEM_SHARED((_N, _H), jnp.float32),
        pltpu.SemaphoreType.DMA,
        pltpu.SemaphoreType.DMA,
    ],
)(_spmm_body)


def kernel(x, edge_index, adj_values, W, bias):
    row = edge_index[0]
    col = edge_index[1]

    xws = _matmul(x, W)

    colr = jnp.concatenate([col, col + _N])
    rowr = row
    adjr = adj_values
    zeros = jnp.zeros((_RPT + 16, _H), dtype=jnp.float32)

    outs = _spmm(xws, colr, rowr, adjr, zeros)

    out = outs.reshape(_NC, _N, _H).transpose(1, 0, 2).reshape(_N, _F_OUT)
    return out + bias
